# baseline (speedup 1.0000x reference)
import jax
import jax.numpy as jnp
from jax import lax
from jax.experimental import pallas as pl
from jax.experimental.pallas import tpu as pltpu
from jax.experimental.pallas import tpu_sc as plsc

DIM = 64
BATCH = 16384
NC = 2
NS = 16
NW = NC * NS
B_PER_W = BATCH // NW


def _body(idx_hbm, w0_hbm, w1_hbm, out0_hbm, out1_hbm,
          idx_v, rows0_v, rows1_v, g0_sem, g1_sem, s0_sem, s1_sem):
    wid = lax.axis_index("s") * NC + lax.axis_index("c")
    base = wid * B_PER_W
    pltpu.sync_copy(idx_hbm.at[pl.ds(base, B_PER_W)], idx_v)
    g0 = pltpu.async_copy(w0_hbm.at[idx_v], rows0_v, g0_sem)
    g1 = pltpu.async_copy(w1_hbm.at[idx_v], rows1_v, g1_sem)
    g0.wait()
    s0 = pltpu.async_copy(rows0_v, out0_hbm.at[pl.ds(base, B_PER_W)], s0_sem)
    g1.wait()
    s1 = pltpu.async_copy(rows1_v, out1_hbm.at[pl.ds(base, B_PER_W)], s1_sem)
    s0.wait()
    s1.wait()


@jax.jit
def _lookup(idx, W0, W1):
    mesh = plsc.VectorSubcoreMesh(core_axis_name="c", subcore_axis_name="s")
    run = pl.kernel(
        _body,
        mesh=mesh,
        out_type=(
            jax.ShapeDtypeStruct((BATCH, DIM), jnp.float32),
            jax.ShapeDtypeStruct((BATCH, DIM), jnp.float32),
        ),
        scratch_types=[
            pltpu.VMEM((B_PER_W,), jnp.int32),
            pltpu.VMEM((B_PER_W, DIM), jnp.float32),
            pltpu.VMEM((B_PER_W, DIM), jnp.float32),
            pltpu.SemaphoreType.DMA,
            pltpu.SemaphoreType.DMA,
            pltpu.SemaphoreType.DMA,
            pltpu.SemaphoreType.DMA,
        ],
        compiler_params=pltpu.CompilerParams(use_tc_tiling_on_sc=False),
    )
    return run(idx, W0, W1)


def kernel(idx, W0, W1):
    e0, e1 = _lookup(idx.astype(jnp.int32), W0, W1)
    return (e0, e1)
